# baseline (speedup 1.0000x reference)
import functools

import jax
import jax.numpy as jnp
from jax import lax
from jax.experimental import pallas as pl
from jax.experimental.pallas import tpu as pltpu
from jax.experimental.pallas import tpu_sc as plsc

D = 64
D2 = 128
B = 16384
L = 16
NC = 2
NS = 16
NW = NC * NS
PER_W = B // NW
CHUNK = 64
NG = CHUNK // L
N_CHUNKS = PER_W // CHUNK
NUM_ENT = 1000000
NUM_REL = 1000


def _make_sc_kernel():
    mesh = plsc.VectorSubcoreMesh(core_axis_name="c", subcore_axis_name="s")

    @functools.partial(
        pl.kernel,
        out_type=jax.ShapeDtypeStruct((NW, L), jnp.float32),
        mesh=mesh,
        compiler_params=pltpu.CompilerParams(needs_layout_passes=False),
        scratch_types=[
            pltpu.VMEM((CHUNK,), jnp.int32),
            pltpu.VMEM((CHUNK,), jnp.int32),
            pltpu.VMEM((CHUNK,), jnp.int32),
            pltpu.VMEM((CHUNK,), jnp.int32),
            pltpu.VMEM((CHUNK,), jnp.int32),
            pltpu.VMEM((CHUNK,), jnp.int32),
            pltpu.VMEM((CHUNK, D2), jnp.float32),
            pltpu.VMEM((CHUNK, D2), jnp.float32),
            pltpu.VMEM((CHUNK, D2), jnp.float32),
            pltpu.VMEM((CHUNK, D2), jnp.float32),
            pltpu.VMEM((CHUNK, D2), jnp.float32),
            pltpu.VMEM((CHUNK, D2), jnp.float32),
            pltpu.VMEM((L,), jnp.float32),
            pltpu.SemaphoreType.DMA,
        ],
    )
    def sc_kernel(heads, rels, tails, ere, eim, rre, rim, out,
                  hidx, ridx, tidx, hdr, rdr, tdr,
                  bhr, bhi, btr, bti, brr, bri, accv, sem):
        wid = lax.axis_index("s") * NC + lax.axis_index("c")
        base = wid * PER_W
        lanes = jax.lax.broadcasted_iota(jnp.int32, (L,), 0)

        def chunk_body(ck, accs):
            off = base + ck * CHUNK
            pltpu.sync_copy(heads.at[pl.ds(off, CHUNK)], hidx)
            pltpu.sync_copy(rels.at[pl.ds(off, CHUNK)], ridx)
            pltpu.sync_copy(tails.at[pl.ds(off, CHUNK)], tidx)
            for g in range(NG):
                sl = pl.ds(g * L, L)
                hdr[sl] = lax.shift_right_logical(hidx[sl], 1)
                rdr[sl] = lax.shift_right_logical(ridx[sl], 1)
                tdr[sl] = lax.shift_right_logical(tidx[sl], 1)
            copies = [
                pltpu.async_copy(ere.at[hdr], bhr, sem),
                pltpu.async_copy(eim.at[hdr], bhi, sem),
                pltpu.async_copy(ere.at[tdr], btr, sem),
                pltpu.async_copy(eim.at[tdr], bti, sem),
                pltpu.async_copy(rre.at[rdr], brr, sem),
                pltpu.async_copy(rim.at[rdr], bri, sem),
            ]
            for c in copies:
                c.wait()

            rows_g, hoff_g, roff_g, toff_g = [], [], [], []
            for g in range(NG):
                sl = pl.ds(g * L, L)
                rows_g.append(lanes + g * L)
                hoff_g.append(lax.bitwise_and(hidx[sl], 1) * D)
                roff_g.append(lax.bitwise_and(ridx[sl], 1) * D)
                toff_g.append(lax.bitwise_and(tidx[sl], 1) * D)

            def d_body(d, a):
                new = []
                for g in range(NG):
                    hc = hoff_g[g] + d
                    rc = roff_g[g] + d
                    tc = toff_g[g] + d
                    vhr = plsc.load_gather(bhr, [rows_g[g], hc])
                    vhi = plsc.load_gather(bhi, [rows_g[g], hc])
                    vtr = plsc.load_gather(btr, [rows_g[g], tc])
                    vti = plsc.load_gather(bti, [rows_g[g], tc])
                    vrr = plsc.load_gather(brr, [rows_g[g], rc])
                    vri = plsc.load_gather(bri, [rows_g[g], rc])
                    new.append(a[g] + vrr * (vhr * vtr + vhi * vti)
                               + vri * (vhr * vti - vhi * vtr))
                return tuple(new)

            return lax.fori_loop(0, D, d_body, accs)

        accs = lax.fori_loop(
            0, N_CHUNKS, chunk_body,
            tuple(jnp.zeros((L,), jnp.float32) for _ in range(NG)))
        total = accs[0]
        for g in range(1, NG):
            total = total + accs[g]
        accv[...] = total
        pltpu.sync_copy(accv, out.at[wid])

    return sc_kernel


_sc_score = _make_sc_kernel()


def kernel(heads, rels, tails, entity_re, entity_im, r_re, r_im):
    parts = _sc_score(
        heads.astype(jnp.int32),
        rels.astype(jnp.int32),
        tails.astype(jnp.int32),
        entity_re.reshape(NUM_ENT // 2, D2),
        entity_im.reshape(NUM_ENT // 2, D2),
        r_re.reshape(NUM_REL // 2, D2),
        r_im.reshape(NUM_REL // 2, D2),
    )
    return jnp.sum(parts)


# speedup vs baseline: 2.1427x; 2.1427x over previous
import functools

import jax
import jax.numpy as jnp
from jax import lax
from jax.experimental import pallas as pl
from jax.experimental.pallas import tpu as pltpu
from jax.experimental.pallas import tpu_sc as plsc

D = 64
B = 16384
L = 16
NC = 2
NS = 16
NW = NC * NS
PER_W = B // NW
CHUNK = 64
NG = CHUNK // L
N_CHUNKS = PER_W // CHUNK
NUM_ENT = 1000000
NUM_REL = 1000


def _make_sc_kernel():
    mesh = plsc.VectorSubcoreMesh(core_axis_name="c", subcore_axis_name="s")

    @functools.partial(
        pl.kernel,
        out_type=jax.ShapeDtypeStruct((NW, L), jnp.float32),
        mesh=mesh,
        compiler_params=pltpu.CompilerParams(needs_layout_passes=False),
        scratch_types=[
            pltpu.VMEM((CHUNK,), jnp.int32),
            pltpu.VMEM((CHUNK,), jnp.int32),
            pltpu.VMEM((CHUNK,), jnp.int32),
            pltpu.VMEM((CHUNK,), jnp.int32),
            pltpu.VMEM((CHUNK, 2 * D), jnp.float32),
            pltpu.VMEM((CHUNK, 2 * D), jnp.float32),
            pltpu.VMEM((L, 8, D), jnp.float32),
            pltpu.VMEM((L, 8, D), jnp.float32),
            pltpu.VMEM((L, 8, D), jnp.float32),
            pltpu.VMEM((L, 8, D), jnp.float32),
            pltpu.VMEM((L,), jnp.float32),
            pltpu.SemaphoreType.DMA,
        ],
    )
    def sc_kernel(heads, rels, tails, ere3, eim3, rre, rim, out,
                  hidx, ridx, tidx, rdr, rbre, rbim,
                  bhre, bhim, btre, btim, accv, sem):
        wid = lax.axis_index("s") * NC + lax.axis_index("c")
        base = wid * PER_W

        def chunk_body(ck, accs):
            off = base + ck * CHUNK
            pltpu.sync_copy(heads.at[pl.ds(off, CHUNK)], hidx)
            pltpu.sync_copy(rels.at[pl.ds(off, CHUNK)], ridx)
            pltpu.sync_copy(tails.at[pl.ds(off, CHUNK)], tidx)
            for g in range(NG):
                sl = pl.ds(g * L, L)
                rdr[sl] = lax.shift_right_logical(ridx[sl], 1)
            cr1 = pltpu.async_copy(rre.at[rdr], rbre, sem)
            cr2 = pltpu.async_copy(rim.at[rdr], rbim, sem)
            cr1.wait()
            cr2.wait()

            def group_body(g, a):
                sl = pl.ds(g * L, L)
                hv = hidx[sl]
                tv = tidx[sl]
                rv = ridx[sl]
                hblk = lax.shift_right_logical(hv, 3)
                tblk = lax.shift_right_logical(tv, 3)
                hsub = lax.bitwise_and(hv, 7)
                tsub = lax.bitwise_and(tv, 7)
                rhalf = lax.bitwise_and(rv, 1) * D

                copies = []
                for l in range(L):
                    copies.append(pltpu.async_copy(
                        ere3.at[hblk[l]], bhre.at[l], sem))
                    copies.append(pltpu.async_copy(
                        eim3.at[hblk[l]], bhim.at[l], sem))
                    copies.append(pltpu.async_copy(
                        ere3.at[tblk[l]], btre.at[l], sem))
                    copies.append(pltpu.async_copy(
                        eim3.at[tblk[l]], btim.at[l], sem))

                new = list(a)
                for l in range(L):
                    for c in copies[4 * l:4 * l + 4]:
                        c.wait()
                    t = g * L + l
                    hs = hsub[l]
                    ts = tsub[l]
                    rh = rhalf[l]
                    for j in range(D // L):
                        dsl = pl.ds(j * L, L)
                        vhr = bhre[l, hs, dsl]
                        vhi = bhim[l, hs, dsl]
                        vtr = btre[l, ts, dsl]
                        vti = btim[l, ts, dsl]
                        rsl = pl.ds(rh + j * L, L)
                        vrr = rbre[t, rsl]
                        vri = rbim[t, rsl]
                        new[j] = (new[j] + vrr * (vhr * vtr + vhi * vti)
                                  + vri * (vhr * vti - vhi * vtr))
                return tuple(new)

            return lax.fori_loop(0, NG, group_body, accs)

        accs = lax.fori_loop(
            0, N_CHUNKS, chunk_body,
            tuple(jnp.zeros((L,), jnp.float32) for _ in range(D // L)))
        total = accs[0]
        for j in range(1, D // L):
            total = total + accs[j]
        accv[...] = total
        pltpu.sync_copy(accv, out.at[wid])

    return sc_kernel


_sc_score = _make_sc_kernel()


def kernel(heads, rels, tails, entity_re, entity_im, r_re, r_im):
    parts = _sc_score(
        heads.astype(jnp.int32),
        rels.astype(jnp.int32),
        tails.astype(jnp.int32),
        entity_re.reshape(NUM_ENT // 8, 8, D),
        entity_im.reshape(NUM_ENT // 8, 8, D),
        r_re.reshape(NUM_REL // 2, 2 * D),
        r_im.reshape(NUM_REL // 2, 2 * D),
    )
    return jnp.sum(parts)
